# Initial kernel scaffold; baseline (speedup 1.0000x reference)
import jax
import jax.numpy as jnp
from jax.experimental import pallas as pl


def kernel(node_feats, edge_feats, edge_index, W_src, b_src, W_dst, b_dst, W_edge, b_edge, gamma_msg, beta_msg, gamma_bn, beta_bn):
    raise NotImplementedError("write your pallas kernel here")


# speedup vs baseline: 2.3523x; 2.3523x over previous
import functools

import jax
import jax.numpy as jnp
from jax import lax
from jax.experimental import pallas as pl
from jax.experimental.pallas import tpu as pltpu
from jax.experimental.pallas import tpu_sc as plsc

N_NODES = 10000
N_EDGES = 320000
F_NODE = 128
F_EDGE = 16
D_MSG = 2 * F_NODE

NW = 32
EPW = N_EDGES // NW
CHUNK = 80
NCH = EPW // CHUNK
N_NODES_PAD = 10240
NPS = N_NODES_PAD // 16

def _proj_body(nf_ref, ws_ref, wd_ref, bs_ref, bd_ref, hs_ref, hd_ref):
    x = nf_ref[...]
    hs_ref[...] = (
        jnp.dot(x, ws_ref[...], preferred_element_type=jnp.float32) + bs_ref[...]
    )
    hd_ref[...] = (
        jnp.dot(x, wd_ref[...], preferred_element_type=jnp.float32) + bd_ref[...]
    )


def _proj(node_feats, ws_t, wd_t, bs, bd):
    blk = 1000
    grid = N_NODES // blk
    return pl.pallas_call(
        _proj_body,
        grid=(grid,),
        in_specs=[
            pl.BlockSpec((blk, F_NODE), lambda i: (i, 0)),
            pl.BlockSpec((F_NODE, D_MSG), lambda i: (0, 0)),
            pl.BlockSpec((F_NODE, D_MSG), lambda i: (0, 0)),
            pl.BlockSpec((1, D_MSG), lambda i: (0, 0)),
            pl.BlockSpec((1, D_MSG), lambda i: (0, 0)),
        ],
        out_specs=[
            pl.BlockSpec((blk, D_MSG), lambda i: (i, 0)),
            pl.BlockSpec((blk, D_MSG), lambda i: (i, 0)),
        ],
        out_shape=[
            jax.ShapeDtypeStruct((N_NODES, D_MSG), jnp.float32),
            jax.ShapeDtypeStruct((N_NODES, D_MSG), jnp.float32),
        ],
    )(node_feats, ws_t, wd_t, bs, bd)


def _gather_body(hsrc, hdst, src, dst, g, idx_s, idx_d, rows, rows2, sem1, sem2):
    c = lax.axis_index("c")
    s = lax.axis_index("s")
    base = (s * 2 + c) * EPW

    def body(k, carry):
        off = pl.multiple_of(base + k * CHUNK, 8)
        pltpu.sync_copy(src.at[pl.ds(off, CHUNK)], idx_s)
        pltpu.sync_copy(dst.at[pl.ds(off, CHUNK)], idx_d)
        cp1 = pltpu.async_copy(hsrc.at[idx_s], rows, sem1)
        cp2 = pltpu.async_copy(hdst.at[idx_d], rows2, sem2)
        cp1.wait()
        cp2.wait()

        def addrow(r, carry2):
            for cc in range(D_MSG // 16):
                sl = pl.ds(cc * 16, 16)
                rows[r, sl] = rows[r, sl] + rows2[r, sl]
            return carry2

        lax.fori_loop(0, CHUNK, addrow, 0)
        pltpu.sync_copy(rows, g.at[pl.ds(off, CHUNK)])
        return carry

    lax.fori_loop(0, NCH, body, 0)


@functools.cache
def _gather_kernel():
    return pl.kernel(
        _gather_body,
        out_type=jax.ShapeDtypeStruct((N_EDGES, D_MSG), jnp.float32),
        mesh=plsc.VectorSubcoreMesh(core_axis_name="c", subcore_axis_name="s"),
        scratch_types=[
            pltpu.VMEM((CHUNK,), jnp.int32),
            pltpu.VMEM((CHUNK,), jnp.int32),
            pltpu.VMEM((CHUNK, D_MSG), jnp.float32),
            pltpu.VMEM((CHUNK, D_MSG), jnp.float32),
            pltpu.SemaphoreType.DMA,
            pltpu.SemaphoreType.DMA,
        ],
    )


def _stats_body(g_ref, ef_ref, we_ref, sums_ref):
    i = pl.program_id(0)
    mpre = g_ref[...] + jnp.dot(
        ef_ref[...], we_ref[...], preferred_element_type=jnp.float32
    )
    s1 = jnp.sum(mpre, axis=0, keepdims=True)
    s2 = jnp.sum(mpre * mpre, axis=0, keepdims=True)
    blk = jnp.concatenate(
        [s1, s2, jnp.zeros((6, D_MSG), jnp.float32)], axis=0
    )

    @pl.when(i == 0)
    def _():
        sums_ref[...] = blk

    @pl.when(i > 0)
    def _():
        sums_ref[...] = sums_ref[...] + blk


def _edge_stats(g, edge_feats, we_t):
    blk = 2000
    grid = N_EDGES // blk
    return pl.pallas_call(
        _stats_body,
        grid=(grid,),
        in_specs=[
            pl.BlockSpec((blk, D_MSG), lambda i: (i, 0)),
            pl.BlockSpec((blk, F_EDGE), lambda i: (i, 0)),
            pl.BlockSpec((F_EDGE, D_MSG), lambda i: (0, 0)),
        ],
        out_specs=pl.BlockSpec((8, D_MSG), lambda i: (0, 0)),
        out_shape=jax.ShapeDtypeStruct((8, D_MSG), jnp.float32),
    )(g, edge_feats, we_t)


def _gate_body(g_ref, ef_ref, we_ref, scale_ref, shift_ref, m_ref):
    mpre = g_ref[...] + jnp.dot(
        ef_ref[...], we_ref[...], preferred_element_type=jnp.float32
    )
    mn = mpre * scale_ref[...] + shift_ref[...]
    h_f = mn[:, :F_NODE]
    h_s = mn[:, F_NODE:]
    m_ref[...] = jax.nn.sigmoid(h_f) * jax.nn.softplus(h_s)


def _edge_gate(g, edge_feats, we_t, scale, shift):
    blk = 2000
    grid = N_EDGES // blk
    return pl.pallas_call(
        _gate_body,
        grid=(grid,),
        in_specs=[
            pl.BlockSpec((blk, D_MSG), lambda i: (i, 0)),
            pl.BlockSpec((blk, F_EDGE), lambda i: (i, 0)),
            pl.BlockSpec((F_EDGE, D_MSG), lambda i: (0, 0)),
            pl.BlockSpec((1, D_MSG), lambda i: (0, 0)),
            pl.BlockSpec((1, D_MSG), lambda i: (0, 0)),
        ],
        out_specs=pl.BlockSpec((blk, F_NODE), lambda i: (i, 0)),
        out_shape=jax.ShapeDtypeStruct((N_EDGES, F_NODE), jnp.float32),
    )(g, edge_feats, we_t, scale, shift)


def _scatter_body(m, dst, out, idx_d, mbuf, zbuf, acc):
    c = lax.axis_index("c")
    s = lax.axis_index("s")
    base = (s * 2 + c) * EPW
    z = jnp.zeros((16,), jnp.float32)

    def zb(r, carry):
        for cc in range(F_NODE // 16):
            zbuf[r, pl.ds(cc * 16, 16)] = z
        return carry

    lax.fori_loop(0, 128, zb, 0)
    for j in range(5):
        pltpu.sync_copy(zbuf, acc.at[pl.ds(s * NPS + j * 128, 128)])
    plsc.subcore_barrier()

    def body(k, carry):
        off = pl.multiple_of(base + k * CHUNK, 8)
        pltpu.sync_copy(dst.at[pl.ds(off, CHUNK)], idx_d)
        pltpu.sync_copy(m.at[pl.ds(off, CHUNK)], mbuf)
        pltpu.sync_copy(mbuf, acc.at[idx_d], add=True)
        return carry

    lax.fori_loop(0, NCH, body, 0)
    plsc.subcore_barrier()
    pltpu.sync_copy(
        acc.at[pl.ds(s * NPS, NPS)],
        out.at[pl.ds(c * N_NODES_PAD + s * NPS, NPS)],
    )


@functools.cache
def _scatter_kernel():
    return pl.kernel(
        _scatter_body,
        out_type=jax.ShapeDtypeStruct((2 * N_NODES_PAD, F_NODE), jnp.float32),
        mesh=plsc.VectorSubcoreMesh(core_axis_name="c", subcore_axis_name="s"),
        scratch_types=[
            pltpu.VMEM((CHUNK,), jnp.int32),
            pltpu.VMEM((CHUNK, F_NODE), jnp.float32),
            pltpu.VMEM((128, F_NODE), jnp.float32),
            pltpu.VMEM_SHARED((N_NODES_PAD, F_NODE), jnp.float32),
        ],
    )


def _final_body(p_ref, nf_ref, gamma_ref, beta_ref, out_ref):
    h = p_ref[0] + p_ref[1]
    mu = jnp.mean(h, axis=0, keepdims=True)
    var = jnp.mean((h - mu) ** 2, axis=0, keepdims=True)
    hb = gamma_ref[...] * (h - mu) / jnp.sqrt(var + 1e-5) + beta_ref[...]
    out_ref[...] = jax.nn.softplus(nf_ref[...] + hb)


def _final(partials, node_feats, gamma, beta):
    return pl.pallas_call(
        _final_body,
        in_specs=[
            pl.BlockSpec((2, N_NODES, F_NODE), lambda: (0, 0, 0)),
            pl.BlockSpec((N_NODES, F_NODE), lambda: (0, 0)),
            pl.BlockSpec((1, F_NODE), lambda: (0, 0)),
            pl.BlockSpec((1, F_NODE), lambda: (0, 0)),
        ],
        out_specs=pl.BlockSpec((N_NODES, F_NODE), lambda: (0, 0)),
        out_shape=jax.ShapeDtypeStruct((N_NODES, F_NODE), jnp.float32),
    )(partials, node_feats, gamma, beta)


def kernel(node_feats, edge_feats, edge_index, W_src, b_src, W_dst, b_dst,
           W_edge, b_edge, gamma_msg, beta_msg, gamma_bn, beta_bn):
    src = edge_index[0]
    dst = edge_index[1]
    h_src, h_dst = _proj(
        node_feats, W_src.T, W_dst.T, b_src.reshape(1, -1), b_dst.reshape(1, -1)
    )
    g = _gather_kernel()(h_src, h_dst, src, dst)
    sums = _edge_stats(g, edge_feats, W_edge.T)
    mean_nob = sums[0] / N_EDGES
    var = sums[1] / N_EDGES - mean_nob * mean_nob
    scale = gamma_msg / jnp.sqrt(var + 1e-5)
    shift = beta_msg - mean_nob * scale
    m = _edge_gate(
        g, edge_feats, W_edge.T, scale.reshape(1, -1), shift.reshape(1, -1)
    )
    partials = _scatter_kernel()(m, dst)
    out = _final(
        partials.reshape(2, N_NODES_PAD, F_NODE)[:, :N_NODES, :],
        node_feats,
        gamma_bn.reshape(1, -1),
        beta_bn.reshape(1, -1),
    )
    return (out, m)


# speedup vs baseline: 3.1976x; 1.3593x over previous
import functools

import jax
import jax.numpy as jnp
from jax import lax
from jax.experimental import pallas as pl
from jax.experimental.pallas import tpu as pltpu
from jax.experimental.pallas import tpu_sc as plsc

N_NODES = 10000
N_EDGES = 320000
F_NODE = 128
F_EDGE = 16
D_MSG = 2 * F_NODE

NW = 32
EPW = N_EDGES // NW
CHUNK = 80
NCH = EPW // CHUNK
N_NODES_PAD = 10240
NPS = N_NODES_PAD // 16

def _proj_body(nf_ref, ws_ref, wd_ref, bs_ref, bd_ref, hs_ref, hd_ref):
    x = nf_ref[...]
    hs_ref[...] = (
        jnp.dot(x, ws_ref[...], preferred_element_type=jnp.float32) + bs_ref[...]
    )
    hd_ref[...] = (
        jnp.dot(x, wd_ref[...], preferred_element_type=jnp.float32) + bd_ref[...]
    )


def _proj(node_feats, ws_t, wd_t, bs, bd):
    blk = 1000
    grid = N_NODES // blk
    return pl.pallas_call(
        _proj_body,
        grid=(grid,),
        in_specs=[
            pl.BlockSpec((blk, F_NODE), lambda i: (i, 0)),
            pl.BlockSpec((F_NODE, D_MSG), lambda i: (0, 0)),
            pl.BlockSpec((F_NODE, D_MSG), lambda i: (0, 0)),
            pl.BlockSpec((1, D_MSG), lambda i: (0, 0)),
            pl.BlockSpec((1, D_MSG), lambda i: (0, 0)),
        ],
        out_specs=[
            pl.BlockSpec((blk, D_MSG), lambda i: (i, 0)),
            pl.BlockSpec((blk, D_MSG), lambda i: (i, 0)),
        ],
        out_shape=[
            jax.ShapeDtypeStruct((N_NODES, D_MSG), jnp.float32),
            jax.ShapeDtypeStruct((N_NODES, D_MSG), jnp.float32),
        ],
    )(node_feats, ws_t, wd_t, bs, bd)


def _gather_body(hsrc, hdst, src3, dst3, g, idxs, idxd,
                 ra, rb, qa, qb, sga, sgb, sqa, sqb, swa, swb):
    c = lax.axis_index("c")
    s = lax.axis_index("s")
    wid = s * 2 + c
    base = wid * EPW
    pltpu.sync_copy(src3.at[wid], idxs)
    pltpu.sync_copy(dst3.at[wid], idxd)

    def issue(k, rows, rows2, ss, sd):
        pltpu.async_copy(hsrc.at[idxs.at[k]], rows, ss)
        pltpu.async_copy(hdst.at[idxd.at[k]], rows2, sd)

    def wait_gather(k, rows, rows2, ss, sd):
        pltpu.make_async_copy(hsrc.at[idxs.at[k]], rows, ss).wait()
        pltpu.make_async_copy(hdst.at[idxd.at[k]], rows2, sd).wait()

    def addrows(rows, rows2):
        def addrow(r, carry2):
            for cc in range(D_MSG // 16):
                sl = pl.ds(cc * 16, 16)
                rows[r, sl] = rows[r, sl] + rows2[r, sl]
            return carry2

        lax.fori_loop(0, CHUNK, addrow, 0)

    def process(k, rows, rows2, ss, sd, sw, n_rows, n_rows2, n_ss, n_sd, n_sw):
        wait_gather(k, rows, rows2, ss, sd)

        @pl.when(k + 1 < NCH)
        def _():
            @pl.when(k >= 1)
            def _():
                off_prev = pl.multiple_of(base + (k - 1) * CHUNK, 8)
                pltpu.make_async_copy(
                    n_rows, g.at[pl.ds(off_prev, CHUNK)], n_sw
                ).wait()

            issue(k + 1, n_rows, n_rows2, n_ss, n_sd)

        addrows(rows, rows2)
        off = pl.multiple_of(base + k * CHUNK, 8)
        pltpu.async_copy(rows, g.at[pl.ds(off, CHUNK)], sw)

    issue(0, ra, qa, sga, sqa)

    def body(k, carry):
        @pl.when(k % 2 == 0)
        def _():
            process(k, ra, qa, sga, sqa, swa, rb, qb, sgb, sqb, swb)

        @pl.when(k % 2 == 1)
        def _():
            process(k, rb, qb, sgb, sqb, swb, ra, qa, sga, sqa, swa)

        return carry

    lax.fori_loop(0, NCH, body, 0)
    off_a = pl.multiple_of(base + (NCH - 1) * CHUNK, 8)
    off_b = pl.multiple_of(base + (NCH - 2) * CHUNK, 8)
    pltpu.make_async_copy(ra, g.at[pl.ds(off_a, CHUNK)], swa).wait()
    pltpu.make_async_copy(rb, g.at[pl.ds(off_b, CHUNK)], swb).wait()


@functools.cache
def _gather_kernel():
    return pl.kernel(
        _gather_body,
        out_type=jax.ShapeDtypeStruct((N_EDGES, D_MSG), jnp.float32),
        mesh=plsc.VectorSubcoreMesh(core_axis_name="c", subcore_axis_name="s"),
        scratch_types=[
            pltpu.VMEM((NCH, CHUNK), jnp.int32),
            pltpu.VMEM((NCH, CHUNK), jnp.int32),
            pltpu.VMEM((CHUNK, D_MSG), jnp.float32),
            pltpu.VMEM((CHUNK, D_MSG), jnp.float32),
            pltpu.VMEM((CHUNK, D_MSG), jnp.float32),
            pltpu.VMEM((CHUNK, D_MSG), jnp.float32),
            pltpu.SemaphoreType.DMA,
            pltpu.SemaphoreType.DMA,
            pltpu.SemaphoreType.DMA,
            pltpu.SemaphoreType.DMA,
            pltpu.SemaphoreType.DMA,
            pltpu.SemaphoreType.DMA,
        ],
    )


def _stats_body(g_ref, ef_ref, we_ref, sums_ref):
    i = pl.program_id(0)
    mpre = g_ref[...] + jnp.dot(
        ef_ref[...], we_ref[...], preferred_element_type=jnp.float32
    )
    s1 = jnp.sum(mpre, axis=0, keepdims=True)
    s2 = jnp.sum(mpre * mpre, axis=0, keepdims=True)
    blk = jnp.concatenate(
        [s1, s2, jnp.zeros((6, D_MSG), jnp.float32)], axis=0
    )

    @pl.when(i == 0)
    def _():
        sums_ref[...] = blk

    @pl.when(i > 0)
    def _():
        sums_ref[...] = sums_ref[...] + blk


def _edge_stats(g, edge_feats, we_t):
    blk = 2000
    grid = N_EDGES // blk
    return pl.pallas_call(
        _stats_body,
        grid=(grid,),
        in_specs=[
            pl.BlockSpec((blk, D_MSG), lambda i: (i, 0)),
            pl.BlockSpec((blk, F_EDGE), lambda i: (i, 0)),
            pl.BlockSpec((F_EDGE, D_MSG), lambda i: (0, 0)),
        ],
        out_specs=pl.BlockSpec((8, D_MSG), lambda i: (0, 0)),
        out_shape=jax.ShapeDtypeStruct((8, D_MSG), jnp.float32),
    )(g, edge_feats, we_t)


def _gate_body(g_ref, ef_ref, we_ref, scale_ref, shift_ref, m_ref):
    mpre = g_ref[...] + jnp.dot(
        ef_ref[...], we_ref[...], preferred_element_type=jnp.float32
    )
    mn = mpre * scale_ref[...] + shift_ref[...]
    h_f = mn[:, :F_NODE]
    h_s = mn[:, F_NODE:]
    m_ref[...] = jax.nn.sigmoid(h_f) * jax.nn.softplus(h_s)


def _edge_gate(g, edge_feats, we_t, scale, shift):
    blk = 2000
    grid = N_EDGES // blk
    return pl.pallas_call(
        _gate_body,
        grid=(grid,),
        in_specs=[
            pl.BlockSpec((blk, D_MSG), lambda i: (i, 0)),
            pl.BlockSpec((blk, F_EDGE), lambda i: (i, 0)),
            pl.BlockSpec((F_EDGE, D_MSG), lambda i: (0, 0)),
            pl.BlockSpec((1, D_MSG), lambda i: (0, 0)),
            pl.BlockSpec((1, D_MSG), lambda i: (0, 0)),
        ],
        out_specs=pl.BlockSpec((blk, F_NODE), lambda i: (i, 0)),
        out_shape=jax.ShapeDtypeStruct((N_EDGES, F_NODE), jnp.float32),
    )(g, edge_feats, we_t, scale, shift)


def _scatter_body(m, dst3, out, idxd, ma, mb, acc, sla, slb, ssa, ssb):
    c = lax.axis_index("c")
    s = lax.axis_index("s")
    wid = s * 2 + c
    base = wid * EPW
    pltpu.sync_copy(dst3.at[wid], idxd)
    z = jnp.zeros((16,), jnp.float32)

    def zb(r, carry):
        for cc in range(F_NODE // 16):
            ma[r, pl.ds(cc * 16, 16)] = z
        return carry

    lax.fori_loop(0, CHUNK, zb, 0)
    for j in range(NPS // CHUNK):
        pltpu.sync_copy(ma, acc.at[pl.ds(s * NPS + j * CHUNK, CHUNK)])
    plsc.subcore_barrier()

    def issue_load(k, mbuf, sl):
        off = pl.multiple_of(base + k * CHUNK, 8)
        pltpu.async_copy(m.at[pl.ds(off, CHUNK)], mbuf, sl)

    def process(k, mbuf, sl, ss, n_mbuf, n_sl, n_ss):
        off = pl.multiple_of(base + k * CHUNK, 8)
        pltpu.make_async_copy(m.at[pl.ds(off, CHUNK)], mbuf, sl).wait()

        @pl.when(k + 1 < NCH)
        def _():
            @pl.when(k >= 1)
            def _():
                pltpu.make_async_copy(
                    n_mbuf, acc.at[idxd.at[k - 1]], n_ss
                ).wait()

            issue_load(k + 1, n_mbuf, n_sl)

        pltpu.async_copy(mbuf, acc.at[idxd.at[k]], ss, add=True)

    issue_load(0, ma, sla)

    def body(k, carry):
        @pl.when(k % 2 == 0)
        def _():
            process(k, ma, sla, ssa, mb, slb, ssb)

        @pl.when(k % 2 == 1)
        def _():
            process(k, mb, slb, ssb, ma, sla, ssa)

        return carry

    lax.fori_loop(0, NCH, body, 0)
    pltpu.make_async_copy(ma, acc.at[idxd.at[NCH - 1]], ssa).wait()
    pltpu.make_async_copy(mb, acc.at[idxd.at[NCH - 2]], ssb).wait()
    plsc.subcore_barrier()
    pltpu.sync_copy(
        acc.at[pl.ds(s * NPS, NPS)],
        out.at[pl.ds(c * N_NODES_PAD + s * NPS, NPS)],
    )


@functools.cache
def _scatter_kernel():
    return pl.kernel(
        _scatter_body,
        out_type=jax.ShapeDtypeStruct((2 * N_NODES_PAD, F_NODE), jnp.float32),
        mesh=plsc.VectorSubcoreMesh(core_axis_name="c", subcore_axis_name="s"),
        scratch_types=[
            pltpu.VMEM((NCH, CHUNK), jnp.int32),
            pltpu.VMEM((CHUNK, F_NODE), jnp.float32),
            pltpu.VMEM((CHUNK, F_NODE), jnp.float32),
            pltpu.VMEM_SHARED((N_NODES_PAD, F_NODE), jnp.float32),
            pltpu.SemaphoreType.DMA,
            pltpu.SemaphoreType.DMA,
            pltpu.SemaphoreType.DMA,
            pltpu.SemaphoreType.DMA,
        ],
    )


def _final_body(p_ref, nf_ref, gamma_ref, beta_ref, out_ref):
    h = p_ref[0] + p_ref[1]
    mu = jnp.mean(h, axis=0, keepdims=True)
    var = jnp.mean((h - mu) ** 2, axis=0, keepdims=True)
    hb = gamma_ref[...] * (h - mu) / jnp.sqrt(var + 1e-5) + beta_ref[...]
    out_ref[...] = jax.nn.softplus(nf_ref[...] + hb)


def _final(partials, node_feats, gamma, beta):
    return pl.pallas_call(
        _final_body,
        in_specs=[
            pl.BlockSpec((2, N_NODES, F_NODE), lambda: (0, 0, 0)),
            pl.BlockSpec((N_NODES, F_NODE), lambda: (0, 0)),
            pl.BlockSpec((1, F_NODE), lambda: (0, 0)),
            pl.BlockSpec((1, F_NODE), lambda: (0, 0)),
        ],
        out_specs=pl.BlockSpec((N_NODES, F_NODE), lambda: (0, 0)),
        out_shape=jax.ShapeDtypeStruct((N_NODES, F_NODE), jnp.float32),
    )(partials, node_feats, gamma, beta)


def kernel(node_feats, edge_feats, edge_index, W_src, b_src, W_dst, b_dst,
           W_edge, b_edge, gamma_msg, beta_msg, gamma_bn, beta_bn):
    src = edge_index[0]
    dst = edge_index[1]
    h_src, h_dst = _proj(
        node_feats, W_src.T, W_dst.T, b_src.reshape(1, -1), b_dst.reshape(1, -1)
    )
    src3 = src.reshape(NW, NCH, CHUNK)
    dst3 = dst.reshape(NW, NCH, CHUNK)
    g = _gather_kernel()(h_src, h_dst, src3, dst3)
    sums = _edge_stats(g, edge_feats, W_edge.T)
    mean_nob = sums[0] / N_EDGES
    var = sums[1] / N_EDGES - mean_nob * mean_nob
    scale = gamma_msg / jnp.sqrt(var + 1e-5)
    shift = beta_msg - mean_nob * scale
    m = _edge_gate(
        g, edge_feats, W_edge.T, scale.reshape(1, -1), shift.reshape(1, -1)
    )
    partials = _scatter_kernel()(m, dst3)
    out = _final(
        partials.reshape(2, N_NODES_PAD, F_NODE)[:, :N_NODES, :],
        node_feats,
        gamma_bn.reshape(1, -1),
        beta_bn.reshape(1, -1),
    )
    return (out, m)
